# baseline (speedup 1.0000x reference)
import functools

import jax
import jax.numpy as jnp
from jax import lax
from jax.experimental import pallas as pl
from jax.experimental.pallas import tpu as pltpu
from jax.experimental.pallas import tpu_sc as plsc

_N = 10000
_E = 320000
_F = 128
_CH = 128
_TCH = _E // _CH
_NC = 2
_NS = 16
_NW = _NC * _NS
_MAXCH = (_TCH + _NW - 1) // _NW
_RPT = (_N // _NS) // 8 * 8
_RTAIL = _N - _NS * _RPT
_BM = 1000
_ILAG = 8
_SLAG = 4

_mesh = plsc.VectorSubcoreMesh(core_axis_name="c", subcore_axis_name="s")


@functools.partial(
    pl.kernel,
    mesh=_mesh,
    out_type=jax.ShapeDtypeStruct((_NC, _N, _F), jnp.float32),
    scratch_types=[
        pltpu.VMEM((_MAXCH, _CH), jnp.int32),
        pltpu.VMEM((_CH, _F), jnp.float32),
        pltpu.VMEM_SHARED((_N, _F), jnp.float32),
        pltpu.SemaphoreType.DMA,
        pltpu.SemaphoreType.DMA,
    ],
)
def _sc_degree(dst_hbm, ones_hbm, zeros_hbm, out_hbm,
               dst_all, ones_v, acc, isem, ssem):
    cid = lax.axis_index("c")
    sid = lax.axis_index("s")
    wid = sid * _NC + cid
    nch = (_TCH - wid + _NW - 1) // _NW

    def iwait_one(j, carry):
        pltpu.make_async_copy(dst_hbm.at[pl.ds(0, _CH)],
                              dst_all.at[0], isem).wait()
        return carry

    def ifire(j, carry):
        off = (wid + j * _NW) * _CH
        pltpu.async_copy(dst_hbm.at[pl.ds(off, _CH)], dst_all.at[j], isem)

        @pl.when(j >= _ILAG)
        def _():
            iwait_one(j, 0)

        return carry

    lax.fori_loop(0, nch, ifire, 0)

    pltpu.sync_copy(ones_hbm, ones_v)
    pltpu.sync_copy(zeros_hbm.at[pl.ds(sid * _RPT, _RPT)],
                    acc.at[pl.ds(sid * _RPT, _RPT)])

    @pl.when(sid == 0)
    def _():
        pltpu.sync_copy(zeros_hbm.at[pl.ds(_NS * _RPT, _RTAIL)],
                        acc.at[pl.ds(_NS * _RPT, _RTAIL)])

    lax.fori_loop(0, _ILAG, iwait_one, 0)
    plsc.subcore_barrier()

    def swait_one(j, carry):
        pltpu.make_async_copy(ones_v, acc.at[dst_all.at[0]], ssem).wait()
        return carry

    def sfire(j, carry):
        pltpu.async_copy(ones_v, acc.at[dst_all.at[j]], ssem, add=True)

        @pl.when(j >= _SLAG)
        def _():
            swait_one(j, 0)

        return carry

    lax.fori_loop(0, nch, sfire, 0)
    lax.fori_loop(0, _SLAG, swait_one, 0)
    plsc.subcore_barrier()

    pltpu.sync_copy(acc.at[pl.ds(sid * _RPT, _RPT)],
                    out_hbm.at[cid, pl.ds(sid * _RPT, _RPT)])

    @pl.when(sid == 0)
    def _():
        pltpu.sync_copy(acc.at[pl.ds(_NS * _RPT, _RTAIL)],
                        out_hbm.at[cid, pl.ds(_NS * _RPT, _RTAIL)])


@functools.partial(
    pl.kernel,
    mesh=_mesh,
    out_type=jax.ShapeDtypeStruct((_NC, _N, _F), jnp.float32),
    scratch_types=[
        pltpu.VMEM((_CH,), jnp.int32),
        pltpu.VMEM((_CH,), jnp.int32),
        pltpu.VMEM((_CH,), jnp.int32),
        pltpu.VMEM((_CH,), jnp.int32),
        pltpu.VMEM((_CH,), jnp.int32),
        pltpu.VMEM((_CH,), jnp.int32),
        pltpu.VMEM((_CH, _F), jnp.float32),
        pltpu.VMEM((_CH, _F), jnp.float32),
        pltpu.VMEM((_CH, _F), jnp.float32),
        pltpu.VMEM_SHARED((_N, _F), jnp.float32),
        pltpu.SemaphoreType.DMA,
        pltpu.SemaphoreType.DMA,
        pltpu.SemaphoreType.DMA,
        pltpu.SemaphoreType.DMA,
        pltpu.SemaphoreType.DMA,
        pltpu.SemaphoreType.DMA,
        pltpu.SemaphoreType.DMA,
        pltpu.SemaphoreType.DMA,
        pltpu.SemaphoreType.DMA,
        pltpu.SemaphoreType.DMA,
        pltpu.SemaphoreType.DMA,
        pltpu.SemaphoreType.DMA,
    ],
)
def _sc_aggregate(p_hbm, src_hbm, dst_hbm, zeros_hbm, out_hbm,
                  sb0, sb1, sb2, db0, db1, db2, rows0, rows1, rows2, acc,
                  isrc0, isrc1, isrc2, idst0, idst1, idst2,
                  gsem0, gsem1, gsem2, ssem0, ssem1, ssem2):
    cid = lax.axis_index("c")
    sid = lax.axis_index("s")
    wid = sid * _NC + cid
    nch = (_TCH - wid + _NW - 1) // _NW
    sb = (sb0, sb1, sb2)
    db = (db0, db1, db2)
    rows = (rows0, rows1, rows2)
    isrc = (isrc0, isrc1, isrc2)
    idst = (idst0, idst1, idst2)
    gsem = (gsem0, gsem1, gsem2)
    ssem = (ssem0, ssem1, ssem2)

    def ioff(j):
        return (wid + j * _NW) * _CH

    def istart_src(j, b):
        pltpu.async_copy(src_hbm.at[pl.ds(ioff(j), _CH)], sb[b], isrc[b])

    def istart_dst(j, b):
        pltpu.async_copy(dst_hbm.at[pl.ds(ioff(j), _CH)], db[b], idst[b])

    def iwait_src(b):
        pltpu.make_async_copy(src_hbm.at[pl.ds(0, _CH)], sb[b],
                              isrc[b]).wait()

    def iwait_dst(b):
        pltpu.make_async_copy(dst_hbm.at[pl.ds(0, _CH)], db[b],
                              idst[b]).wait()

    def gstart(j, b):
        pltpu.async_copy(p_hbm.at[sb[b]], rows[b], gsem[b])

    def gwait(b):
        pltpu.make_async_copy(p_hbm.at[sb[b]], rows[b], gsem[b]).wait()

    def sstart(j, b):
        pltpu.async_copy(rows[b], acc.at[db[b]], ssem[b], add=True)

    def swait(b):
        pltpu.make_async_copy(rows[b], acc.at[db[b]], ssem[b]).wait()

    istart_src(0, 0)
    istart_src(1, 1)
    istart_src(2, 2)
    istart_dst(0, 0)
    istart_dst(1, 1)

    pltpu.sync_copy(zeros_hbm.at[pl.ds(sid * _RPT, _RPT)],
                    acc.at[pl.ds(sid * _RPT, _RPT)])

    @pl.when(sid == 0)
    def _():
        pltpu.sync_copy(zeros_hbm.at[pl.ds(_NS * _RPT, _RTAIL)],
                        acc.at[pl.ds(_NS * _RPT, _RTAIL)])

    plsc.subcore_barrier()

    iwait_src(0)
    gstart(0, 0)
    iwait_src(1)
    gstart(1, 1)

    def body(t, carry):
        j0 = 3 * t

        def unit(u):
            j = j0 + u
            b = u
            b2 = (u + 2) % 3

            @pl.when(j + 2 < nch)
            def _():
                @pl.when(j >= 1)
                def _():
                    swait(b2)

                istart_dst(j + 2, b2)
                iwait_src(b2)
                gstart(j + 2, b2)

            gwait(b)

            @pl.when(j + 3 < nch)
            def _():
                istart_src(j + 3, b)

            iwait_dst(b)
            sstart(j, b)

        unit(0)
        unit(1)
        unit(2)
        return carry

    lax.fori_loop(0, 26, body, 0)

    @pl.when(nch % 3 == 1)
    def _():
        gwait(0)
        iwait_dst(0)
        sstart(_MAXCH - 1, 0)

    swait(0)
    swait(1)
    swait(2)
    plsc.subcore_barrier()
    pltpu.sync_copy(acc.at[pl.ds(sid * _RPT, _RPT)],
                    out_hbm.at[cid, pl.ds(sid * _RPT, _RPT)])

    @pl.when(sid == 0)
    def _():
        pltpu.sync_copy(acc.at[pl.ds(_NS * _RPT, _RTAIL)],
                        out_hbm.at[cid, pl.ds(_NS * _RPT, _RTAIL)])


def _norm_from(d0, d1):
    deg = d0[:, 0:1] + d1[:, 0:1]
    return jnp.where(deg > 0.0, lax.rsqrt(jnp.maximum(deg, 1.0)), 0.0)


def _tc_mm_body(x_ref, w_ref, o_ref):
    o_ref[...] = jnp.dot(x_ref[...], w_ref[...],
                         preferred_element_type=jnp.float32)


def _tc_scale_body(q_ref, d0_ref, d1_ref, o_ref):
    o_ref[...] = q_ref[...] * _norm_from(d0_ref[...], d1_ref[...])


def _tc_mid_body(a0_ref, a1_ref, d0_ref, d1_ref, b_ref, w_ref, o_ref):
    norm = _norm_from(d0_ref[...], d1_ref[...])
    h = jnp.maximum((a0_ref[...] + a1_ref[...]) * norm + b_ref[...], 0.0)
    o_ref[...] = jnp.dot(h, w_ref[...],
                         preferred_element_type=jnp.float32) * norm


def _tc_out_body(a0_ref, a1_ref, d0_ref, d1_ref, b_ref, w_ref, bo_ref, o_ref):
    norm = _norm_from(d0_ref[...], d1_ref[...])
    h = jnp.maximum((a0_ref[...] + a1_ref[...]) * norm + b_ref[...], 0.0)
    o_ref[...] = jnp.dot(h, w_ref[...],
                         preferred_element_type=jnp.float32) + bo_ref[...]


def _row_spec(w):
    return pl.BlockSpec((_BM, w), lambda i: (i, 0))


def _full_spec(shape):
    return pl.BlockSpec(shape, lambda i: (0, 0))


def kernel(x, edge_index, W1, b1, W2, b2, W_out, b_out):
    src = edge_index[0]
    dst = edge_index[1]
    nclass = W_out.shape[1]
    grid = (_N // _BM,)

    ones128 = jnp.ones((_CH, _F), jnp.float32)
    zerosf = jnp.zeros((_N, _F), jnp.float32)

    q1 = pl.pallas_call(
        _tc_mm_body,
        grid=grid,
        in_specs=[_row_spec(_F), _full_spec((_F, _F))],
        out_specs=_row_spec(_F),
        out_shape=jax.ShapeDtypeStruct((_N, _F), jnp.float32),
    )(x, W1)

    degp = _sc_degree(dst, ones128, zerosf)
    d0, d1 = degp[0], degp[1]

    p1 = pl.pallas_call(
        _tc_scale_body,
        grid=grid,
        in_specs=[_row_spec(_F), _row_spec(_F), _row_spec(_F)],
        out_specs=_row_spec(_F),
        out_shape=jax.ShapeDtypeStruct((_N, _F), jnp.float32),
    )(q1, d0, d1)

    a1 = _sc_aggregate(p1, src, dst, zerosf)

    p2 = pl.pallas_call(
        _tc_mid_body,
        grid=grid,
        in_specs=[_row_spec(_F), _row_spec(_F), _row_spec(_F), _row_spec(_F),
                  _full_spec((1, _F)), _full_spec((_F, _F))],
        out_specs=_row_spec(_F),
        out_shape=jax.ShapeDtypeStruct((_N, _F), jnp.float32),
    )(a1[0], a1[1], d0, d1, b1.reshape(1, _F), W2)

    a2 = _sc_aggregate(p2, src, dst, zerosf)

    out = pl.pallas_call(
        _tc_out_body,
        grid=grid,
        in_specs=[_row_spec(_F), _row_spec(_F), _row_spec(_F), _row_spec(_F),
                  _full_spec((1, _F)), _full_spec((_F, nclass)),
                  _full_spec((1, nclass))],
        out_specs=_row_spec(nclass),
        out_shape=jax.ShapeDtypeStruct((_N, nclass), jnp.float32),
    )(a2[0], a2[1], d0, d1, b2.reshape(1, _F), W_out, b_out.reshape(1, nclass))

    return out


# speedup vs baseline: 1.0031x; 1.0031x over previous
import functools

import jax
import jax.numpy as jnp
from jax import lax
from jax.experimental import pallas as pl
from jax.experimental.pallas import tpu as pltpu
from jax.experimental.pallas import tpu_sc as plsc

_N = 10000
_E = 320000
_F = 128
_CH = 128
_TCH = _E // _CH
_NC = 2
_NS = 16
_NW = _NC * _NS
_MAXCH = (_TCH + _NW - 1) // _NW
_RPT = (_N // _NS) // 8 * 8
_RTAIL = _N - _NS * _RPT
_BM = 1000
_ILAG = 8
_SLAG = 4

_mesh = plsc.VectorSubcoreMesh(core_axis_name="c", subcore_axis_name="s")


@functools.partial(
    pl.kernel,
    mesh=_mesh,
    out_type=jax.ShapeDtypeStruct((_NC, _N, _F), jnp.float32),
    scratch_types=[
        pltpu.VMEM((_MAXCH, _CH), jnp.int32),
        pltpu.VMEM((_CH, _F), jnp.float32),
        pltpu.VMEM_SHARED((_N, _F), jnp.float32),
        pltpu.SemaphoreType.DMA,
        pltpu.SemaphoreType.DMA,
    ],
)
def _sc_degree(dst_hbm, ones_hbm, zeros_hbm, out_hbm,
               dst_all, ones_v, acc, isem, ssem):
    cid = lax.axis_index("c")
    sid = lax.axis_index("s")
    wid = sid * _NC + cid
    nch = (_TCH - wid + _NW - 1) // _NW

    def iwait_one(j, carry):
        pltpu.make_async_copy(dst_hbm.at[pl.ds(0, _CH)],
                              dst_all.at[0], isem).wait()
        return carry

    def ifire(j, carry):
        off = (wid + j * _NW) * _CH
        pltpu.async_copy(dst_hbm.at[pl.ds(off, _CH)], dst_all.at[j], isem)

        @pl.when(j >= _ILAG)
        def _():
            iwait_one(j, 0)

        return carry

    lax.fori_loop(0, nch, ifire, 0)

    pltpu.sync_copy(ones_hbm, ones_v)
    pltpu.sync_copy(zeros_hbm.at[pl.ds(sid * _RPT, _RPT)],
                    acc.at[pl.ds(sid * _RPT, _RPT)])

    @pl.when(sid == 0)
    def _():
        pltpu.sync_copy(zeros_hbm.at[pl.ds(_NS * _RPT, _RTAIL)],
                        acc.at[pl.ds(_NS * _RPT, _RTAIL)])

    lax.fori_loop(0, _ILAG, iwait_one, 0)
    plsc.subcore_barrier()

    def swait_one(j, carry):
        pltpu.make_async_copy(ones_v, acc.at[dst_all.at[0]], ssem).wait()
        return carry

    def sfire(j, carry):
        pltpu.async_copy(ones_v, acc.at[dst_all.at[j]], ssem, add=True)

        @pl.when(j >= _SLAG)
        def _():
            swait_one(j, 0)

        return carry

    lax.fori_loop(0, nch, sfire, 0)
    lax.fori_loop(0, _SLAG, swait_one, 0)
    plsc.subcore_barrier()

    pltpu.sync_copy(acc.at[pl.ds(sid * _RPT, _RPT)],
                    out_hbm.at[cid, pl.ds(sid * _RPT, _RPT)])

    @pl.when(sid == 0)
    def _():
        pltpu.sync_copy(acc.at[pl.ds(_NS * _RPT, _RTAIL)],
                        out_hbm.at[cid, pl.ds(_NS * _RPT, _RTAIL)])


@functools.partial(
    pl.kernel,
    mesh=_mesh,
    out_type=jax.ShapeDtypeStruct((_NC, _N, _F), jnp.float32),
    scratch_types=[
        pltpu.VMEM((_CH,), jnp.int32),
        pltpu.VMEM((_CH,), jnp.int32),
        pltpu.VMEM((_CH,), jnp.int32),
        pltpu.VMEM((_CH,), jnp.int32),
        pltpu.VMEM((_CH,), jnp.int32),
        pltpu.VMEM((_CH,), jnp.int32),
        pltpu.VMEM((_CH, _F), jnp.float32),
        pltpu.VMEM((_CH, _F), jnp.float32),
        pltpu.VMEM((_CH, _F), jnp.float32),
        pltpu.VMEM_SHARED((_N, _F), jnp.float32),
        pltpu.SemaphoreType.DMA,
        pltpu.SemaphoreType.DMA,
        pltpu.SemaphoreType.DMA,
        pltpu.SemaphoreType.DMA,
        pltpu.SemaphoreType.DMA,
        pltpu.SemaphoreType.DMA,
        pltpu.SemaphoreType.DMA,
        pltpu.SemaphoreType.DMA,
        pltpu.SemaphoreType.DMA,
        pltpu.SemaphoreType.DMA,
        pltpu.SemaphoreType.DMA,
        pltpu.SemaphoreType.DMA,
    ],
)
def _sc_aggregate(p_hbm, src_hbm, dst_hbm, zeros_hbm, out_hbm,
                  sb0, sb1, sb2, db0, db1, db2, rows0, rows1, rows2, acc,
                  isrc0, isrc1, isrc2, idst0, idst1, idst2,
                  gsem0, gsem1, gsem2, ssem0, ssem1, ssem2):
    cid = lax.axis_index("c")
    sid = lax.axis_index("s")
    wid = sid * _NC + cid
    nch = (_TCH - wid + _NW - 1) // _NW
    sb = (sb0, sb1, sb2)
    db = (db0, db1, db2)
    rows = (rows0, rows1, rows2)
    isrc = (isrc0, isrc1, isrc2)
    idst = (idst0, idst1, idst2)
    gsem = (gsem0, gsem1, gsem2)
    ssem = (ssem0, ssem1, ssem2)

    def ioff(j):
        return (wid + j * _NW) * _CH

    def istart_src(j, b):
        pltpu.async_copy(src_hbm.at[pl.ds(ioff(j), _CH)], sb[b], isrc[b])

    def istart_dst(j, b):
        pltpu.async_copy(dst_hbm.at[pl.ds(ioff(j), _CH)], db[b], idst[b])

    def iwait_src(b):
        pltpu.make_async_copy(src_hbm.at[pl.ds(0, _CH)], sb[b],
                              isrc[b]).wait()

    def iwait_dst(b):
        pltpu.make_async_copy(dst_hbm.at[pl.ds(0, _CH)], db[b],
                              idst[b]).wait()

    def gstart(j, b):
        pltpu.async_copy(p_hbm.at[sb[b]], rows[b], gsem[b])

    def gwait(b):
        pltpu.make_async_copy(p_hbm.at[sb[b]], rows[b], gsem[b]).wait()

    def sstart(j, b):
        pltpu.async_copy(rows[b], acc.at[db[b]], ssem[b], add=True)

    def swait(b):
        pltpu.make_async_copy(rows[b], acc.at[db[b]], ssem[b]).wait()

    istart_src(0, 0)
    istart_src(1, 1)
    istart_src(2, 2)
    istart_dst(0, 0)
    istart_dst(1, 1)

    pltpu.sync_copy(zeros_hbm.at[pl.ds(sid * _RPT, _RPT)],
                    acc.at[pl.ds(sid * _RPT, _RPT)])

    @pl.when(sid == 0)
    def _():
        pltpu.sync_copy(zeros_hbm.at[pl.ds(_NS * _RPT, _RTAIL)],
                        acc.at[pl.ds(_NS * _RPT, _RTAIL)])

    plsc.subcore_barrier()

    iwait_src(0)
    gstart(0, 0)
    iwait_src(1)
    gstart(1, 1)

    def body(t, carry):
        j0 = 3 * t

        def unit(u):
            j = j0 + u
            b = u
            b2 = (u + 2) % 3

            @pl.when(j + 2 < nch)
            def _():
                @pl.when(j >= 1)
                def _():
                    swait(b2)

                istart_dst(j + 2, b2)
                iwait_src(b2)
                gstart(j + 2, b2)

            gwait(b)

            @pl.when(j + 3 < nch)
            def _():
                istart_src(j + 3, b)

            iwait_dst(b)
            sstart(j, b)

        unit(0)
        unit(1)
        unit(2)
        return carry

    lax.fori_loop(0, 26, body, 0)

    @pl.when(nch % 3 == 1)
    def _():
        gwait(0)
        iwait_dst(0)
        sstart(_MAXCH - 1, 0)

    swait(0)
    swait(1)
    swait(2)
    plsc.subcore_barrier()
    pltpu.sync_copy(acc.at[pl.ds(sid * _RPT, _RPT)],
                    out_hbm.at[cid, pl.ds(sid * _RPT, _RPT)])

    @pl.when(sid == 0)
    def _():
        pltpu.sync_copy(acc.at[pl.ds(_NS * _RPT, _RTAIL)],
                        out_hbm.at[cid, pl.ds(_NS * _RPT, _RTAIL)])


def _norm_from(d0, d1):
    deg = d0[:, 0:1] + d1[:, 0:1]
    return jnp.where(deg > 0.0, lax.rsqrt(jnp.maximum(deg, 1.0)), 0.0)


def _tc_mm1_body(x_ref, w_ref, d0_ref, d1_ref, o_ref):
    norm = _norm_from(d0_ref[...], d1_ref[...])
    o_ref[...] = jnp.dot(x_ref[...], w_ref[...],
                         preferred_element_type=jnp.float32) * norm


def _tc_mid_body(a0_ref, a1_ref, d0_ref, d1_ref, b_ref, w_ref, o_ref):
    norm = _norm_from(d0_ref[...], d1_ref[...])
    h = jnp.maximum((a0_ref[...] + a1_ref[...]) * norm + b_ref[...], 0.0)
    o_ref[...] = jnp.dot(h, w_ref[...],
                         preferred_element_type=jnp.float32) * norm


def _tc_out_body(a0_ref, a1_ref, d0_ref, d1_ref, b_ref, w_ref, bo_ref, o_ref):
    norm = _norm_from(d0_ref[...], d1_ref[...])
    h = jnp.maximum((a0_ref[...] + a1_ref[...]) * norm + b_ref[...], 0.0)
    o_ref[...] = jnp.dot(h, w_ref[...],
                         preferred_element_type=jnp.float32) + bo_ref[...]


def _row_spec(w):
    return pl.BlockSpec((_BM, w), lambda i: (i, 0))


def _full_spec(shape):
    return pl.BlockSpec(shape, lambda i: (0, 0))


def kernel(x, edge_index, W1, b1, W2, b2, W_out, b_out):
    src = edge_index[0]
    dst = edge_index[1]
    nclass = W_out.shape[1]
    grid = (_N // _BM,)

    ones128 = jnp.ones((_CH, _F), jnp.float32)
    zerosf = jnp.zeros((_N, _F), jnp.float32)

    degp = _sc_degree(dst, ones128, zerosf)
    d0, d1 = degp[0], degp[1]

    p1 = pl.pallas_call(
        _tc_mm1_body,
        grid=grid,
        in_specs=[_row_spec(_F), _full_spec((_F, _F)),
                  _row_spec(_F), _row_spec(_F)],
        out_specs=_row_spec(_F),
        out_shape=jax.ShapeDtypeStruct((_N, _F), jnp.float32),
    )(x, W1, d0, d1)

    a1 = _sc_aggregate(p1, src, dst, zerosf)

    p2 = pl.pallas_call(
        _tc_mid_body,
        grid=grid,
        in_specs=[_row_spec(_F), _row_spec(_F), _row_spec(_F), _row_spec(_F),
                  _full_spec((1, _F)), _full_spec((_F, _F))],
        out_specs=_row_spec(_F),
        out_shape=jax.ShapeDtypeStruct((_N, _F), jnp.float32),
    )(a1[0], a1[1], d0, d1, b1.reshape(1, _F), W2)

    a2 = _sc_aggregate(p2, src, dst, zerosf)

    out = pl.pallas_call(
        _tc_out_body,
        grid=grid,
        in_specs=[_row_spec(_F), _row_spec(_F), _row_spec(_F), _row_spec(_F),
                  _full_spec((1, _F)), _full_spec((_F, nclass)),
                  _full_spec((1, nclass))],
        out_specs=_row_spec(nclass),
        out_shape=jax.ShapeDtypeStruct((_N, nclass), jnp.float32),
    )(a2[0], a2[1], d0, d1, b2.reshape(1, _F), W_out, b_out.reshape(1, nclass))

    return out
